# initial kernel scaffold (unmeasured)
import jax
import jax.numpy as jnp
from jax import lax
from jax.experimental import pallas as pl
from jax.experimental.pallas import tpu as pltpu

N_DEV = 8
B, SQ, D = 2, 256, 768
HL, DH = 8, 64
FLAT = B * SQ
HDL = HL * DH


def kernel(x, Wq, Wo, Wk, Wv):
    def body(x_ref, wq_ref, wo_ref, wk_ref, wv_ref, out_ref,
             attn_ref, comm_ref, send_sems, recv_sems):
        my = lax.axis_index("i")
        left = lax.rem(my + N_DEV - 1, N_DEV)
        right = lax.rem(my + 1, N_DEV)

        barrier_sem = pltpu.get_barrier_semaphore()
        for nbr in (left, right):
            pl.semaphore_signal(
                barrier_sem, inc=1,
                device_id=(nbr,), device_id_type=pl.DeviceIdType.MESH,
            )
        pl.semaphore_wait(barrier_sem, 2)

        xb = x_ref[...].reshape(FLAT, D).astype(jnp.bfloat16)
        q = jnp.dot(xb, wq_ref[...].astype(jnp.bfloat16),
                    preferred_element_type=jnp.float32)
        k = jnp.dot(xb, wk_ref[...].astype(jnp.bfloat16),
                    preferred_element_type=jnp.float32)
        v = jnp.dot(xb, wv_ref[...].astype(jnp.bfloat16),
                    preferred_element_type=jnp.float32)
        for b in range(B):
            rows = pl.ds(b * SQ, SQ)
            for h in range(HL):
                cols = pl.ds(h * DH, DH)
                qh = q[rows, cols].astype(jnp.bfloat16)
                kh = k[rows, cols].astype(jnp.bfloat16)
                vh = v[rows, cols].astype(jnp.bfloat16)
                s = jnp.dot(qh, kh.T, preferred_element_type=jnp.float32)
                s = s * 0.125
                m = jnp.max(s, axis=-1, keepdims=True)
                p = jnp.exp(s - m)
                l = jnp.sum(p, axis=-1, keepdims=True)
                o = jnp.dot(p.astype(jnp.bfloat16), vh,
                            preferred_element_type=jnp.float32) / l
                attn_ref[rows, cols] = o.astype(jnp.bfloat16)
        partial = jnp.dot(attn_ref[...], wo_ref[...].astype(jnp.bfloat16),
                          preferred_element_type=jnp.float32)

        comm_ref[0] = partial
        acc = partial
        for hop in range(N_DEV - 1):
            rdma = pltpu.make_async_remote_copy(
                src_ref=comm_ref.at[hop],
                dst_ref=comm_ref.at[hop + 1],
                send_sem=send_sems.at[hop],
                recv_sem=recv_sems.at[hop],
                device_id=(right,),
                device_id_type=pl.DeviceIdType.MESH,
            )
            rdma.start()
            rdma.wait()
            acc = acc + comm_ref[hop + 1]
        out_ref[...] = acc.reshape(B, SQ, D)

    return pl.pallas_call(
        body,
        out_shape=jax.ShapeDtypeStruct((B, SQ, D), jnp.float32),
        in_specs=[pl.BlockSpec(memory_space=pltpu.VMEM)] * 5,
        out_specs=pl.BlockSpec(memory_space=pltpu.VMEM),
        scratch_shapes=[
            pltpu.VMEM((FLAT, HDL), jnp.bfloat16),
            pltpu.VMEM((N_DEV, FLAT, D), jnp.float32),
            pltpu.SemaphoreType.DMA((N_DEV - 1,)),
            pltpu.SemaphoreType.DMA((N_DEV - 1,)),
        ],
        compiler_params=pltpu.CompilerParams(collective_id=0),
    )(x, Wq, Wk, Wv, Wo)


def _body_arg_order_note():
    pass


# baseline (device time: 150739 ns/iter reference)
import jax
import jax.numpy as jnp
from jax import lax
from jax.experimental import pallas as pl
from jax.experimental.pallas import tpu as pltpu

N_DEV = 8
B, SQ, D = 2, 256, 768
HL, DH = 8, 64
FLAT = B * SQ
HDL = HL * DH


def kernel(x, Wq, Wo, Wk, Wv):
    def body(x_ref, wq_ref, wk_ref, wv_ref, wo_ref, out_ref,
             attn_ref, comm_ref, send_sems, recv_sems):
        my = lax.axis_index("i")
        left = lax.rem(my + N_DEV - 1, N_DEV)
        right = lax.rem(my + 1, N_DEV)

        barrier_sem = pltpu.get_barrier_semaphore()
        for nbr in (left, right):
            pl.semaphore_signal(
                barrier_sem, inc=1,
                device_id=(nbr,), device_id_type=pl.DeviceIdType.MESH,
            )
        pl.semaphore_wait(barrier_sem, 2)

        xb = x_ref[...].reshape(FLAT, D).astype(jnp.bfloat16)
        q = jnp.dot(xb, wq_ref[...].astype(jnp.bfloat16),
                    preferred_element_type=jnp.float32)
        k = jnp.dot(xb, wk_ref[...].astype(jnp.bfloat16),
                    preferred_element_type=jnp.float32)
        v = jnp.dot(xb, wv_ref[...].astype(jnp.bfloat16),
                    preferred_element_type=jnp.float32)
        for b in range(B):
            rows = slice(b * SQ, (b + 1) * SQ)
            for h in range(HL):
                cols = slice(h * DH, (h + 1) * DH)
                qh = q[rows, cols].astype(jnp.bfloat16)
                kh = k[rows, cols].astype(jnp.bfloat16)
                vh = v[rows, cols].astype(jnp.bfloat16)
                s = jnp.dot(qh, kh.T, preferred_element_type=jnp.float32)
                s = s * 0.125
                m = jnp.max(s, axis=-1, keepdims=True)
                p = jnp.exp(s - m)
                l = jnp.sum(p, axis=-1, keepdims=True)
                o = jnp.dot(p.astype(jnp.bfloat16), vh,
                            preferred_element_type=jnp.float32) / l
                attn_ref[rows, cols] = o.astype(jnp.bfloat16)
        partial = jnp.dot(attn_ref[...], wo_ref[...].astype(jnp.bfloat16),
                          preferred_element_type=jnp.float32)

        comm_ref[0] = partial
        acc = partial
        for hop in range(N_DEV - 1):
            rdma = pltpu.make_async_remote_copy(
                src_ref=comm_ref.at[hop],
                dst_ref=comm_ref.at[hop + 1],
                send_sem=send_sems.at[hop],
                recv_sem=recv_sems.at[hop],
                device_id=(right,),
                device_id_type=pl.DeviceIdType.MESH,
            )
            rdma.start()
            rdma.wait()
            acc = acc + comm_ref[hop + 1]
        out_ref[...] = acc.reshape(B, SQ, D)

    return pl.pallas_call(
        body,
        out_shape=jax.ShapeDtypeStruct((B, SQ, D), jnp.float32),
        in_specs=[pl.BlockSpec(memory_space=pltpu.VMEM)] * 5,
        out_specs=pl.BlockSpec(memory_space=pltpu.VMEM),
        scratch_shapes=[
            pltpu.VMEM((FLAT, HDL), jnp.bfloat16),
            pltpu.VMEM((N_DEV, FLAT, D), jnp.float32),
            pltpu.SemaphoreType.DMA((N_DEV - 1,)),
            pltpu.SemaphoreType.DMA((N_DEV - 1,)),
        ],
        compiler_params=pltpu.CompilerParams(collective_id=0),
    )(x, Wq, Wk, Wv, Wo)


# device time: 45257 ns/iter; 3.3307x vs baseline; 3.3307x over previous
import jax
import jax.numpy as jnp
from jax import lax
from jax.experimental import pallas as pl
from jax.experimental.pallas import tpu as pltpu

N_DEV = 8
B, SQ, D = 2, 256, 768
HL, DH = 8, 64
FLAT = B * SQ
HDL = HL * DH


def kernel(x, Wq, Wo, Wk, Wv):
    def body(x_ref, wq_ref, wk_ref, wv_ref, wo_ref, out_ref,
             attn_ref, work_ref, rs_recv_ref, send_sems, recv_sems):
        my = lax.axis_index("i")

        barrier_sem = pltpu.get_barrier_semaphore()
        for s in range(3):
            pl.semaphore_signal(
                barrier_sem, inc=1,
                device_id=(my ^ (1 << s),),
                device_id_type=pl.DeviceIdType.MESH,
            )
        pl.semaphore_wait(barrier_sem, 3)

        xb = x_ref[...].reshape(FLAT, D).astype(jnp.bfloat16)
        q = jnp.dot(xb, wq_ref[...].astype(jnp.bfloat16),
                    preferred_element_type=jnp.float32)
        k = jnp.dot(xb, wk_ref[...].astype(jnp.bfloat16),
                    preferred_element_type=jnp.float32)
        v = jnp.dot(xb, wv_ref[...].astype(jnp.bfloat16),
                    preferred_element_type=jnp.float32)
        for b in range(B):
            rows = slice(b * SQ, (b + 1) * SQ)
            for h in range(HL):
                cols = slice(h * DH, (h + 1) * DH)
                qh = q[rows, cols].astype(jnp.bfloat16)
                kh = k[rows, cols].astype(jnp.bfloat16)
                vh = v[rows, cols].astype(jnp.bfloat16)
                s = jnp.dot(qh, kh.T, preferred_element_type=jnp.float32)
                s = s * 0.125
                m = jnp.max(s, axis=-1, keepdims=True)
                p = jnp.exp(s - m)
                l = jnp.sum(p, axis=-1, keepdims=True)
                o = jnp.dot(p.astype(jnp.bfloat16), vh,
                            preferred_element_type=jnp.float32) / l
                attn_ref[rows, cols] = o.astype(jnp.bfloat16)
        partial = jnp.dot(attn_ref[...], wo_ref[...].astype(jnp.bfloat16),
                          preferred_element_type=jnp.float32)

        work_ref[...] = partial.astype(jnp.bfloat16)

        pending = []
        start = 0
        size = FLAT
        for s in range(3):
            half = size // 2
            bit = (my >> s) & 1
            partner = my ^ (1 << s)
            keep_start = start + bit * half
            send_start = start + (1 - bit) * half
            off = {0: 0, 1: 256, 2: 384}[s]
            rdma = pltpu.make_async_remote_copy(
                src_ref=work_ref.at[pl.ds(send_start, half)],
                dst_ref=rs_recv_ref.at[pl.ds(off, half)],
                send_sem=send_sems.at[s],
                recv_sem=recv_sems.at[s],
                device_id=(partner,),
                device_id_type=pl.DeviceIdType.MESH,
            )
            rdma.start()
            rdma.wait_recv()
            pending.append(rdma)
            mine = work_ref[pl.ds(keep_start, half), :].astype(jnp.float32)
            theirs = rs_recv_ref[off:off + half, :].astype(jnp.float32)
            work_ref[pl.ds(keep_start, half), :] = (
                (mine + theirs).astype(jnp.bfloat16))
            start = keep_start
            size = half

        for s in (2, 1, 0):
            bit = (my >> s) & 1
            partner = my ^ (1 << s)
            rdma = pltpu.make_async_remote_copy(
                src_ref=work_ref.at[pl.ds(start, size)],
                dst_ref=work_ref.at[pl.ds(start, size)],
                send_sem=send_sems.at[5 - s],
                recv_sem=recv_sems.at[5 - s],
                device_id=(partner,),
                device_id_type=pl.DeviceIdType.MESH,
            )
            rdma.start()
            rdma.wait_recv()
            pending.append(rdma)
            start = start - bit * size
            size = size * 2

        out_ref[...] = work_ref[...].astype(jnp.float32).reshape(B, SQ, D)
        for rdma in pending:
            rdma.wait_send()

    return pl.pallas_call(
        body,
        out_shape=jax.ShapeDtypeStruct((B, SQ, D), jnp.float32),
        in_specs=[pl.BlockSpec(memory_space=pltpu.VMEM)] * 5,
        out_specs=pl.BlockSpec(memory_space=pltpu.VMEM),
        scratch_shapes=[
            pltpu.VMEM((FLAT, HDL), jnp.bfloat16),
            pltpu.VMEM((FLAT, D), jnp.bfloat16),
            pltpu.VMEM((448, D), jnp.bfloat16),
            pltpu.SemaphoreType.DMA((6,)),
            pltpu.SemaphoreType.DMA((6,)),
        ],
        compiler_params=pltpu.CompilerParams(collective_id=0),
    )(x, Wq, Wk, Wv, Wo)


# device time: 43566 ns/iter; 3.4600x vs baseline; 1.0388x over previous
import jax
import jax.numpy as jnp
from jax import lax
from jax.experimental import pallas as pl
from jax.experimental.pallas import tpu as pltpu

N_DEV = 8
B, SQ, D = 2, 256, 768
HL, DH = 8, 64
FLAT = B * SQ
HDL = HL * DH


def kernel(x, Wq, Wo, Wk, Wv):
    def body(x_ref, wq_ref, wk_ref, wv_ref, wo_ref, out_ref,
             attn_ref, work_ref, rs_recv_ref, send_sems, recv_sems):
        my = lax.axis_index("i")

        masks = (1, 3, 4)
        bits = (
            (my ^ (my >> 1)) & 1,
            (my >> 1) & 1,
            (my >> 2) & 1,
        )

        barrier_sem = pltpu.get_barrier_semaphore()
        for mask in masks:
            pl.semaphore_signal(
                barrier_sem, inc=1,
                device_id=(my ^ mask,),
                device_id_type=pl.DeviceIdType.MESH,
            )
        pl.semaphore_wait(barrier_sem, 3)

        xb = x_ref[...].reshape(FLAT, D).astype(jnp.bfloat16)
        q = jnp.dot(xb, wq_ref[...].astype(jnp.bfloat16),
                    preferred_element_type=jnp.float32)
        k = jnp.dot(xb, wk_ref[...].astype(jnp.bfloat16),
                    preferred_element_type=jnp.float32)
        v = jnp.dot(xb, wv_ref[...].astype(jnp.bfloat16),
                    preferred_element_type=jnp.float32)
        for b in range(B):
            rows = slice(b * SQ, (b + 1) * SQ)
            for h in range(HL):
                cols = slice(h * DH, (h + 1) * DH)
                qh = q[rows, cols].astype(jnp.bfloat16)
                kh = k[rows, cols].astype(jnp.bfloat16)
                vh = v[rows, cols].astype(jnp.bfloat16)
                s = jnp.dot(qh, kh.T, preferred_element_type=jnp.float32)
                s = s * 0.125
                m = jnp.max(s, axis=-1, keepdims=True)
                p = jnp.exp(s - m)
                l = jnp.sum(p, axis=-1, keepdims=True)
                o = jnp.dot(p.astype(jnp.bfloat16), vh,
                            preferred_element_type=jnp.float32) / l
                attn_ref[rows, cols] = o.astype(jnp.bfloat16)
        partial = jnp.dot(attn_ref[...], wo_ref[...].astype(jnp.bfloat16),
                          preferred_element_type=jnp.float32)

        work_ref[...] = partial.astype(jnp.bfloat16)

        pending = []
        start = 0
        size = FLAT
        for s in range(3):
            half = size // 2
            bit = bits[s]
            partner = my ^ masks[s]
            keep_start = start + bit * half
            send_start = start + (1 - bit) * half
            off = {0: 0, 1: 256, 2: 384}[s]
            rdma = pltpu.make_async_remote_copy(
                src_ref=work_ref.at[pl.ds(send_start, half)],
                dst_ref=rs_recv_ref.at[pl.ds(off, half)],
                send_sem=send_sems.at[s],
                recv_sem=recv_sems.at[s],
                device_id=(partner,),
                device_id_type=pl.DeviceIdType.MESH,
            )
            rdma.start()
            rdma.wait_recv()
            pending.append(rdma)
            mine = work_ref[pl.ds(keep_start, half), :].astype(jnp.float32)
            theirs = rs_recv_ref[off:off + half, :].astype(jnp.float32)
            work_ref[pl.ds(keep_start, half), :] = (
                (mine + theirs).astype(jnp.bfloat16))
            start = keep_start
            size = half

        for s in (2, 1, 0):
            bit = bits[s]
            partner = my ^ masks[s]
            rdma = pltpu.make_async_remote_copy(
                src_ref=work_ref.at[pl.ds(start, size)],
                dst_ref=work_ref.at[pl.ds(start, size)],
                send_sem=send_sems.at[5 - s],
                recv_sem=recv_sems.at[5 - s],
                device_id=(partner,),
                device_id_type=pl.DeviceIdType.MESH,
            )
            rdma.start()
            rdma.wait_recv()
            pending.append(rdma)
            start = start - bit * size
            size = size * 2

        out_ref[...] = work_ref[...].astype(jnp.float32).reshape(B, SQ, D)
        for rdma in pending:
            rdma.wait_send()

    return pl.pallas_call(
        body,
        out_shape=jax.ShapeDtypeStruct((B, SQ, D), jnp.float32),
        in_specs=[pl.BlockSpec(memory_space=pltpu.VMEM)] * 5,
        out_specs=pl.BlockSpec(memory_space=pltpu.VMEM),
        scratch_shapes=[
            pltpu.VMEM((FLAT, HDL), jnp.bfloat16),
            pltpu.VMEM((FLAT, D), jnp.bfloat16),
            pltpu.VMEM((448, D), jnp.bfloat16),
            pltpu.SemaphoreType.DMA((6,)),
            pltpu.SemaphoreType.DMA((6,)),
        ],
        compiler_params=pltpu.CompilerParams(collective_id=0),
    )(x, Wq, Wk, Wv, Wo)


# device time: 37244 ns/iter; 4.0473x vs baseline; 1.1697x over previous
import jax
import jax.numpy as jnp
from jax import lax
from jax.experimental import pallas as pl
from jax.experimental.pallas import tpu as pltpu

N_DEV = 8
B, SQ, D = 2, 256, 768
HL, DH = 8, 64
FLAT = B * SQ
HDL = HL * DH


def kernel(x, Wq, Wo, Wk, Wv):
    def body(x_ref, wq_ref, wk_ref, wv_ref, wo_ref, out_ref,
             attn_ref, work_ref, rs_recv_ref, send_sems, recv_sems):
        my = lax.axis_index("i")

        b0 = my & 1
        b1 = (my >> 1) & 1
        b2 = (my >> 2) & 1
        masks_a = (1, 3, 4)
        bits_a = (b0 ^ b1, b1, b2)
        masks_b = (3, 4, 1)
        bits_b = (b1, b2, b0)

        barrier_sem = pltpu.get_barrier_semaphore()
        for mask in masks_a:
            pl.semaphore_signal(
                barrier_sem, inc=1,
                device_id=(my ^ mask,),
                device_id_type=pl.DeviceIdType.MESH,
            )
        pl.semaphore_wait(barrier_sem, 3)

        xb = x_ref[...].reshape(FLAT, D).astype(jnp.bfloat16)
        q = jnp.dot(xb, wq_ref[...].astype(jnp.bfloat16),
                    preferred_element_type=jnp.float32)
        k = jnp.dot(xb, wk_ref[...].astype(jnp.bfloat16),
                    preferred_element_type=jnp.float32)
        v = jnp.dot(xb, wv_ref[...].astype(jnp.bfloat16),
                    preferred_element_type=jnp.float32)
        for b in range(B):
            rows = slice(b * SQ, (b + 1) * SQ)
            for h in range(HL):
                cols = slice(h * DH, (h + 1) * DH)
                qh = q[rows, cols].astype(jnp.bfloat16)
                kh = k[rows, cols].astype(jnp.bfloat16)
                vh = v[rows, cols].astype(jnp.bfloat16)
                s = jnp.dot(qh, kh.T, preferred_element_type=jnp.float32)
                s = s * 0.125
                m = jnp.max(s, axis=-1, keepdims=True)
                p = jnp.exp(s - m)
                l = jnp.sum(p, axis=-1, keepdims=True)
                o = jnp.dot(p.astype(jnp.bfloat16), vh,
                            preferred_element_type=jnp.float32) / l
                attn_ref[rows, cols] = o.astype(jnp.bfloat16)
        partial = jnp.dot(attn_ref[...], wo_ref[...].astype(jnp.bfloat16),
                          preferred_element_type=jnp.float32)

        work_ref[...] = partial.astype(jnp.bfloat16)

        HC = D // 2
        cols_of = (slice(0, HC), slice(HC, D))
        pending = []
        sem_i = iter(range(12))
        rs_off = (0, 256, 384)

        def exchange(src_rows, dst_ref, dst_rows, ci, partner):
            i = next(sem_i)
            rdma = pltpu.make_async_remote_copy(
                src_ref=work_ref.at[src_rows, cols_of[ci]],
                dst_ref=dst_ref.at[dst_rows, cols_of[ci]],
                send_sem=send_sems.at[i],
                recv_sem=recv_sems.at[i],
                device_id=(partner,),
                device_id_type=pl.DeviceIdType.MESH,
            )
            rdma.start()
            return rdma

        starts = [0, 0]
        size = FLAT
        for s in range(3):
            half = size // 2
            rd = []
            keeps = []
            for ci, (msk, bit) in enumerate(
                    ((masks_a[s], bits_a[s]), (masks_b[s], bits_b[s]))):
                keep = starts[ci] + bit * half
                send = starts[ci] + (1 - bit) * half
                rd.append(exchange(pl.ds(send, half), rs_recv_ref,
                                   pl.ds(rs_off[s], half), ci, my ^ msk))
                keeps.append(keep)
            for ci in range(2):
                rd[ci].wait_recv()
                pending.append(rd[ci])
                cs = cols_of[ci]
                mine = work_ref[pl.ds(keeps[ci], half), cs].astype(jnp.float32)
                theirs = rs_recv_ref[
                    rs_off[s]:rs_off[s] + half, cs].astype(jnp.float32)
                work_ref[pl.ds(keeps[ci], half), cs] = (
                    (mine + theirs).astype(jnp.bfloat16))
                starts[ci] = keeps[ci]
            size = half

        for s in (2, 1, 0):
            rd = []
            for ci, (msk, bit) in enumerate(
                    ((masks_a[s], bits_a[s]), (masks_b[s], bits_b[s]))):
                rows = pl.ds(starts[ci], size)
                rd.append(exchange(rows, work_ref, rows, ci, my ^ msk))
                starts[ci] = starts[ci] - bit * size
            for ci in range(2):
                rd[ci].wait_recv()
                pending.append(rd[ci])
            size = size * 2

        out_ref[...] = work_ref[...].astype(jnp.float32).reshape(B, SQ, D)
        for rdma in pending:
            rdma.wait_send()

    return pl.pallas_call(
        body,
        out_shape=jax.ShapeDtypeStruct((B, SQ, D), jnp.float32),
        in_specs=[pl.BlockSpec(memory_space=pltpu.VMEM)] * 5,
        out_specs=pl.BlockSpec(memory_space=pltpu.VMEM),
        scratch_shapes=[
            pltpu.VMEM((FLAT, HDL), jnp.bfloat16),
            pltpu.VMEM((FLAT, D), jnp.bfloat16),
            pltpu.VMEM((448, D), jnp.bfloat16),
            pltpu.SemaphoreType.DMA((12,)),
            pltpu.SemaphoreType.DMA((12,)),
        ],
        compiler_params=pltpu.CompilerParams(collective_id=0),
    )(x, Wq, Wk, Wv, Wo)


# device time: 36202 ns/iter; 4.1638x vs baseline; 1.0288x over previous
import jax
import jax.numpy as jnp
from jax import lax
from jax.experimental import pallas as pl
from jax.experimental.pallas import tpu as pltpu

N_DEV = 8
B, SQ, D = 2, 256, 768
HL, DH = 8, 64
FLAT = B * SQ
HDL = HL * DH
HC = D // 2


def kernel(x, Wq, Wo, Wk, Wv):
    def body(x_hbm, wq_hbm, wk_hbm, wv_hbm, wo_hbm, out_ref,
             x_v, wq_v, wk_v, wv_v, wo_v,
             attn_ref, work_ref, rs_recv_ref,
             copy_sems, send_sems, recv_sems):
        my = lax.axis_index("i")
        b0 = my & 1
        b1 = (my >> 1) & 1
        b2 = (my >> 2) & 1
        masks_a = (1, 3, 4)
        bits_a = (b0 ^ b1, b1, b2)
        masks_b = (3, 4, 1)
        bits_b = (b1, b2, b0)
        cols_of = (slice(0, HC), slice(HC, D))

        copies = []
        for i, (src, dst) in enumerate(
                ((x_hbm, x_v), (wq_hbm, wq_v), (wk_hbm, wk_v),
                 (wv_hbm, wv_v), (wo_hbm, wo_v))):
            c = pltpu.make_async_copy(src, dst, copy_sems.at[i])
            c.start()
            copies.append(c)

        barrier_sem = pltpu.get_barrier_semaphore()
        for mask in masks_a:
            pl.semaphore_signal(
                barrier_sem, inc=1,
                device_id=(my ^ mask,),
                device_id_type=pl.DeviceIdType.MESH,
            )

        copies[0].wait()
        copies[1].wait()
        xb = x_v[...].reshape(FLAT, D).astype(jnp.bfloat16)
        q = jnp.dot(xb, wq_v[...].astype(jnp.bfloat16),
                    preferred_element_type=jnp.float32)
        copies[2].wait()
        k = jnp.dot(xb, wk_v[...].astype(jnp.bfloat16),
                    preferred_element_type=jnp.float32)
        copies[3].wait()
        v = jnp.dot(xb, wv_v[...].astype(jnp.bfloat16),
                    preferred_element_type=jnp.float32)
        for b in range(B):
            rows = slice(b * SQ, (b + 1) * SQ)
            for h in range(HL):
                cols = slice(h * DH, (h + 1) * DH)
                qh = q[rows, cols].astype(jnp.bfloat16)
                kh = k[rows, cols].astype(jnp.bfloat16)
                vh = v[rows, cols].astype(jnp.bfloat16)
                s = jnp.dot(qh, kh.T, preferred_element_type=jnp.float32)
                s = s * 0.125
                m = jnp.max(s, axis=-1, keepdims=True)
                p = jnp.exp(s - m)
                l = jnp.sum(p, axis=-1, keepdims=True)
                o = jnp.dot(p.astype(jnp.bfloat16), vh,
                            preferred_element_type=jnp.float32) / l
                attn_ref[rows, cols] = o.astype(jnp.bfloat16)

        pending = []
        sem_i = iter(range(12))
        rs_off = (0, 256, 384)
        rs_half = (256, 128, 64)
        starts = [0, 0]
        cur = [FLAT, FLAT]

        def exchange(src_rows, dst_ref, dst_rows, ci, partner):
            i = next(sem_i)
            rdma = pltpu.make_async_remote_copy(
                src_ref=work_ref.at[src_rows, cols_of[ci]],
                dst_ref=dst_ref.at[dst_rows, cols_of[ci]],
                send_sem=send_sems.at[i],
                recv_sem=recv_sems.at[i],
                device_id=(partner,),
                device_id_type=pl.DeviceIdType.MESH,
            )
            rdma.start()
            return rdma

        def stage_of(ci, s):
            return (masks_a[s], bits_a[s]) if ci == 0 else \
                (masks_b[s], bits_b[s])

        def rs_start(ci, s):
            msk, bit = stage_of(ci, s)
            half = rs_half[s]
            keep = starts[ci] + bit * half
            send = starts[ci] + (1 - bit) * half
            rdma = exchange(pl.ds(send, half), rs_recv_ref,
                            pl.ds(rs_off[s], half), ci, my ^ msk)
            return rdma, keep, half, s

        def rs_finish(ci, st):
            rdma, keep, half, s = st
            rdma.wait_recv()
            pending.append(rdma)
            cs = cols_of[ci]
            mine = work_ref[pl.ds(keep, half), cs].astype(jnp.float32)
            theirs = rs_recv_ref[
                rs_off[s]:rs_off[s] + half, cs].astype(jnp.float32)
            work_ref[pl.ds(keep, half), cs] = (
                (mine + theirs).astype(jnp.bfloat16))
            starts[ci] = keep
            cur[ci] = half

        def ag_start(ci, s):
            msk, bit = stage_of(ci, s)
            size = cur[ci]
            rows = pl.ds(starts[ci], size)
            rdma = exchange(rows, work_ref, rows, ci, my ^ msk)
            pstart = starts[ci] + (1 - 2 * bit) * size
            starts[ci] = starts[ci] - bit * size
            cur[ci] = 2 * size
            return rdma, pstart, size

        def ag_finish(ci, st):
            rdma, pstart, size = st
            rdma.wait_recv()
            pending.append(rdma)
            cs = cols_of[ci]
            out_ref[pl.ds(pstart, size), cs] = (
                work_ref[pl.ds(pstart, size), cs].astype(jnp.float32))

        copies[4].wait()
        wo_bf = wo_v[...].astype(jnp.bfloat16)
        attn = attn_ref[...]
        work_ref[:, cols_of[0]] = jnp.dot(
            attn, wo_bf[:, cols_of[0]],
            preferred_element_type=jnp.float32).astype(jnp.bfloat16)
        pl.semaphore_wait(barrier_sem, 3)
        a = rs_start(0, 0)
        work_ref[:, cols_of[1]] = jnp.dot(
            attn, wo_bf[:, cols_of[1]],
            preferred_element_type=jnp.float32).astype(jnp.bfloat16)
        b = rs_start(1, 0)
        rs_finish(0, a)
        a = rs_start(0, 1)
        rs_finish(1, b)
        b = rs_start(1, 1)
        rs_finish(0, a)
        a = rs_start(0, 2)
        rs_finish(1, b)
        b = rs_start(1, 2)
        rs_finish(0, a)
        out_ref[pl.ds(starts[0], 64), cols_of[0]] = (
            work_ref[pl.ds(starts[0], 64), cols_of[0]].astype(jnp.float32))
        a = ag_start(0, 2)
        rs_finish(1, b)
        out_ref[pl.ds(starts[1], 64), cols_of[1]] = (
            work_ref[pl.ds(starts[1], 64), cols_of[1]].astype(jnp.float32))
        b = ag_start(1, 2)
        ag_finish(0, a)
        a = ag_start(0, 1)
        ag_finish(1, b)
        b = ag_start(1, 1)
        ag_finish(0, a)
        a = ag_start(0, 0)
        ag_finish(1, b)
        b = ag_start(1, 0)
        ag_finish(0, a)
        ag_finish(1, b)

        for rdma in pending:
            rdma.wait_send()

    flat = pl.pallas_call(
        body,
        out_shape=jax.ShapeDtypeStruct((FLAT, D), jnp.float32),
        in_specs=[pl.BlockSpec(memory_space=pl.ANY)] * 5,
        out_specs=pl.BlockSpec(memory_space=pltpu.VMEM),
        scratch_shapes=[
            pltpu.VMEM((B, SQ, D), jnp.float32),
            pltpu.VMEM((D, HDL), jnp.float32),
            pltpu.VMEM((D, HDL), jnp.float32),
            pltpu.VMEM((D, HDL), jnp.float32),
            pltpu.VMEM((HDL, D), jnp.float32),
            pltpu.VMEM((FLAT, HDL), jnp.bfloat16),
            pltpu.VMEM((FLAT, D), jnp.bfloat16),
            pltpu.VMEM((448, D), jnp.bfloat16),
            pltpu.SemaphoreType.DMA((5,)),
            pltpu.SemaphoreType.DMA((12,)),
            pltpu.SemaphoreType.DMA((12,)),
        ],
        compiler_params=pltpu.CompilerParams(collective_id=0),
    )(x, Wq, Wk, Wv, Wo)
    return flat.reshape(B, SQ, D)


# device time: 32693 ns/iter; 4.6107x vs baseline; 1.1073x over previous
import jax
import jax.numpy as jnp
from jax import lax
from jax.experimental import pallas as pl
from jax.experimental.pallas import tpu as pltpu

N_DEV = 8
B, SQ, D = 2, 256, 768
HL, DH = 8, 64
FLAT = B * SQ
HDL = HL * DH
TC = D // 3


def kernel(x, Wq, Wo, Wk, Wv):
    def body(x_hbm, wq_hbm, wk_hbm, wv_hbm, wo_hbm, out_ref,
             x_v, wq_v, wk_v, wv_v, wo_v,
             attn_ref, work_ref, rs_recv_ref,
             copy_sems, send_sems, recv_sems):
        my = lax.axis_index("i")
        b0 = my & 1
        b1 = (my >> 1) & 1
        b2 = (my >> 2) & 1
        stage_sets = (
            ((1, b0 ^ b1), (3, b1), (4, b2)),
            ((3, b1), (4, b2), (1, b0)),
            ((4, b2), (1, b0 ^ b1), (3, b1)),
        )
        cols_of = (slice(0, TC), slice(TC, 2 * TC), slice(2 * TC, D))

        copies = []
        for i, (src, dst) in enumerate(
                ((x_hbm, x_v), (wq_hbm, wq_v), (wk_hbm, wk_v),
                 (wv_hbm, wv_v), (wo_hbm, wo_v))):
            c = pltpu.make_async_copy(src, dst, copy_sems.at[i])
            c.start()
            copies.append(c)

        barrier_sem = pltpu.get_barrier_semaphore()
        for mask in (1, 3, 4):
            pl.semaphore_signal(
                barrier_sem, inc=1,
                device_id=(my ^ mask,),
                device_id_type=pl.DeviceIdType.MESH,
            )

        copies[0].wait()
        copies[1].wait()
        xb = x_v[...].reshape(FLAT, D).astype(jnp.bfloat16)
        q = jnp.dot(xb, wq_v[...].astype(jnp.bfloat16),
                    preferred_element_type=jnp.float32)
        copies[2].wait()
        k = jnp.dot(xb, wk_v[...].astype(jnp.bfloat16),
                    preferred_element_type=jnp.float32)
        copies[3].wait()
        v = jnp.dot(xb, wv_v[...].astype(jnp.bfloat16),
                    preferred_element_type=jnp.float32)
        for b in range(B):
            rows = slice(b * SQ, (b + 1) * SQ)
            for h in range(HL):
                cols = slice(h * DH, (h + 1) * DH)
                qh = q[rows, cols].astype(jnp.bfloat16)
                kh = k[rows, cols].astype(jnp.bfloat16)
                vh = v[rows, cols].astype(jnp.bfloat16)
                s = jnp.dot(qh, kh.T, preferred_element_type=jnp.float32)
                p = jnp.exp(s * 0.125)
                l = jnp.sum(p, axis=-1, keepdims=True)
                o = jnp.dot(p.astype(jnp.bfloat16), vh,
                            preferred_element_type=jnp.float32) / l
                attn_ref[rows, cols] = o.astype(jnp.bfloat16)

        pending = []
        sem_i = iter(range(18))
        rs_off = (0, 256, 384)
        rs_half = (256, 128, 64)
        starts = [0, 0, 0]
        cur = [FLAT, FLAT, FLAT]

        def exchange(src_rows, dst_ref, dst_rows, ci, partner):
            i = next(sem_i)
            rdma = pltpu.make_async_remote_copy(
                src_ref=work_ref.at[src_rows, cols_of[ci]],
                dst_ref=dst_ref.at[dst_rows, cols_of[ci]],
                send_sem=send_sems.at[i],
                recv_sem=recv_sems.at[i],
                device_id=(partner,),
                device_id_type=pl.DeviceIdType.MESH,
            )
            rdma.start()
            return rdma

        def stage_of(ci, s):
            return stage_sets[ci][s]

        def rs_start(ci, s):
            msk, bit = stage_of(ci, s)
            half = rs_half[s]
            keep = starts[ci] + bit * half
            send = starts[ci] + (1 - bit) * half
            rdma = exchange(pl.ds(send, half), rs_recv_ref,
                            pl.ds(rs_off[s], half), ci, my ^ msk)
            return rdma, keep, half, s

        def rs_finish(ci, st):
            rdma, keep, half, s = st
            rdma.wait_recv()
            pending.append(rdma)
            cs = cols_of[ci]
            mine = work_ref[pl.ds(keep, half), cs].astype(jnp.float32)
            theirs = rs_recv_ref[
                rs_off[s]:rs_off[s] + half, cs].astype(jnp.float32)
            work_ref[pl.ds(keep, half), cs] = (
                (mine + theirs).astype(jnp.bfloat16))
            starts[ci] = keep
            cur[ci] = half

        def ag_start(ci, s):
            msk, bit = stage_of(ci, s)
            size = cur[ci]
            rows = pl.ds(starts[ci], size)
            rdma = exchange(rows, work_ref, rows, ci, my ^ msk)
            pstart = starts[ci] + (1 - 2 * bit) * size
            starts[ci] = starts[ci] - bit * size
            cur[ci] = 2 * size
            return rdma, pstart, size

        def ag_finish(ci, st):
            rdma, pstart, size = st
            rdma.wait_recv()
            pending.append(rdma)
            cs = cols_of[ci]
            out_ref[pl.ds(pstart, size), cs] = (
                work_ref[pl.ds(pstart, size), cs].astype(jnp.float32))

        copies[4].wait()
        wo_bf = wo_v[...].astype(jnp.bfloat16)
        attn = attn_ref[...]

        def outproj(ci):
            work_ref[:, cols_of[ci]] = jnp.dot(
                attn, wo_bf[:, cols_of[ci]],
                preferred_element_type=jnp.float32).astype(jnp.bfloat16)

        outproj(0)
        pl.semaphore_wait(barrier_sem, 3)
        st = [None, None, None]
        st[0] = rs_start(0, 0)
        outproj(1)
        st[1] = rs_start(1, 0)
        outproj(2)
        st[2] = rs_start(2, 0)
        for s in (1, 2):
            for ci in range(3):
                rs_finish(ci, st[ci])
                st[ci] = rs_start(ci, s)
        for ci in range(3):
            rs_finish(ci, st[ci])
            out_ref[pl.ds(starts[ci], 64), cols_of[ci]] = (
                work_ref[pl.ds(starts[ci], 64),
                         cols_of[ci]].astype(jnp.float32))
            st[ci] = ag_start(ci, 2)
        for s in (1, 0):
            for ci in range(3):
                ag_finish(ci, st[ci])
                st[ci] = ag_start(ci, s)
        for ci in range(3):
            ag_finish(ci, st[ci])

        for rdma in pending:
            rdma.wait_send()

    flat = pl.pallas_call(
        body,
        out_shape=jax.ShapeDtypeStruct((FLAT, D), jnp.float32),
        in_specs=[pl.BlockSpec(memory_space=pl.ANY)] * 5,
        out_specs=pl.BlockSpec(memory_space=pltpu.VMEM),
        scratch_shapes=[
            pltpu.VMEM((B, SQ, D), jnp.float32),
            pltpu.VMEM((D, HDL), jnp.float32),
            pltpu.VMEM((D, HDL), jnp.float32),
            pltpu.VMEM((D, HDL), jnp.float32),
            pltpu.VMEM((HDL, D), jnp.float32),
            pltpu.VMEM((FLAT, HDL), jnp.bfloat16),
            pltpu.VMEM((FLAT, D), jnp.bfloat16),
            pltpu.VMEM((448, D), jnp.bfloat16),
            pltpu.SemaphoreType.DMA((5,)),
            pltpu.SemaphoreType.DMA((18,)),
            pltpu.SemaphoreType.DMA((18,)),
        ],
        compiler_params=pltpu.CompilerParams(collective_id=0),
    )(x, Wq, Wk, Wv, Wo)
    return flat.reshape(B, SQ, D)


# device time: 26083 ns/iter; 5.7792x vs baseline; 1.2534x over previous
import jax
import jax.numpy as jnp
from jax import lax
from jax.experimental import pallas as pl
from jax.experimental.pallas import tpu as pltpu

N_DEV = 8
B, SQ, D = 2, 256, 768
HL, DH = 8, 64
FLAT = B * SQ
HDL = HL * DH
TC = D // 3


def kernel(x, Wq, Wo, Wk, Wv):
    def body(x_hbm, wq_hbm, wk_hbm, wv_hbm, wo_hbm, out_ref,
             x_v, wq_v, wk_v, wv_v, wo_v,
             attn_ref, work_ref, recv_ref,
             copy_sems, send_sems, recv_sems):
        my = lax.axis_index("i")
        b0 = my & 1
        b1 = (my >> 1) & 1
        b2 = (my >> 2) & 1
        masks = (
            (1, 3, 4),
            (3, 4, 1),
            (4, 1, 3),
        )
        cols_of = (slice(0, TC), slice(TC, 2 * TC), slice(2 * TC, D))

        copies = []
        for i, (src, dst) in enumerate(
                ((x_hbm, x_v), (wq_hbm, wq_v), (wk_hbm, wk_v),
                 (wv_hbm, wv_v), (wo_hbm, wo_v))):
            c = pltpu.make_async_copy(src, dst, copy_sems.at[i])
            c.start()
            copies.append(c)

        barrier_sem = pltpu.get_barrier_semaphore()
        for mask in (1, 3, 4):
            pl.semaphore_signal(
                barrier_sem, inc=1,
                device_id=(my ^ mask,),
                device_id_type=pl.DeviceIdType.MESH,
            )

        copies[0].wait()
        copies[1].wait()
        xb = x_v[...].reshape(FLAT, D).astype(jnp.bfloat16)
        wq_bf = wq_v[...].astype(jnp.bfloat16)

        def qkv(b):
            rows = slice(b * SQ, (b + 1) * SQ)
            qb = jnp.dot(xb[rows], wq_bf,
                         preferred_element_type=jnp.float32)
            kb = jnp.dot(xb[rows], wk_bf,
                         preferred_element_type=jnp.float32)
            vb = jnp.dot(xb[rows], wv_bf,
                         preferred_element_type=jnp.float32)
            return qb, kb, vb

        def attention(b, qb, kb, vb):
            rows = slice(b * SQ, (b + 1) * SQ)
            ss = []
            for h in range(HL):
                cols = slice(h * DH, (h + 1) * DH)
                qh = qb[:, cols].astype(jnp.bfloat16)
                kh = kb[:, cols].astype(jnp.bfloat16)
                ss.append(jnp.dot(qh, kh.T,
                                  preferred_element_type=jnp.float32))
            p = jnp.exp(jnp.concatenate(ss, axis=1) * 0.125)
            for h in range(HL):
                cols = slice(h * DH, (h + 1) * DH)
                ph = p[:, h * SQ:(h + 1) * SQ]
                vh = vb[:, cols].astype(jnp.bfloat16)
                l = jnp.sum(ph, axis=-1, keepdims=True)
                o = jnp.dot(ph.astype(jnp.bfloat16), vh,
                            preferred_element_type=jnp.float32) / l
                attn_ref[rows, cols] = o.astype(jnp.bfloat16)


        QR = FLAT // 4

        def exchange(ci, w, r):
            i = 12 * w + 3 * r + ci
            rows = slice(r * QR, (r + 1) * QR)
            rdma = pltpu.make_async_remote_copy(
                src_ref=work_ref.at[rows, cols_of[ci]],
                dst_ref=recv_ref.at[w, rows, cols_of[ci]],
                send_sem=send_sems.at[i],
                recv_sem=recv_sems.at[i],
                device_id=(my ^ masks[ci][w],),
                device_id_type=pl.DeviceIdType.MESH,
            )
            rdma.start()
            return rdma

        def outproj(ci, r):
            rows = slice(r * QR, (r + 1) * QR)
            work_ref[rows, cols_of[ci]] = jnp.dot(
                attn_ref[rows, :], wo_bf[:, cols_of[ci]],
                preferred_element_type=jnp.float32).astype(jnp.bfloat16)

        qb = jnp.dot(xb[0:SQ], wq_bf, preferred_element_type=jnp.float32)
        copies[2].wait()
        wk_bf = wk_v[...].astype(jnp.bfloat16)
        kb = jnp.dot(xb[0:SQ], wk_bf, preferred_element_type=jnp.float32)
        copies[3].wait()
        wv_bf = wv_v[...].astype(jnp.bfloat16)
        vb = jnp.dot(xb[0:SQ], wv_bf, preferred_element_type=jnp.float32)
        attention(0, qb, kb, vb)
        copies[4].wait()
        wo_bf = wo_v[...].astype(jnp.bfloat16)

        st = [[None] * 3 for _ in range(4)]
        outproj(0, 0)
        pl.semaphore_wait(barrier_sem, 3)
        st[0][0] = exchange(0, 0, 0)
        for ci in (1, 2):
            outproj(ci, 0)
            st[0][ci] = exchange(ci, 0, 0)
        for ci in range(3):
            outproj(ci, 1)
            st[1][ci] = exchange(ci, 0, 1)

        qb, kb, vb = qkv(1)
        attention(1, qb, kb, vb)
        for r in (2, 3):
            for ci in range(3):
                outproj(ci, r)
                st[r][ci] = exchange(ci, 0, r)

        for w in range(3):
            for r in range(4):
                rows = slice(r * QR, (r + 1) * QR)
                for ci in range(3):
                    st[r][ci].wait_recv()
                    st[r][ci].wait_send()
                    cs = cols_of[ci]
                    work_ref[rows, cs] = (
                        work_ref[rows, cs] + recv_ref[w, rows, cs])
                    if w < 2:
                        st[r][ci] = exchange(ci, w + 1, r)
                    else:
                        out_ref[rows, cs] = (
                            work_ref[rows, cs].astype(jnp.float32))

    flat = pl.pallas_call(
        body,
        out_shape=jax.ShapeDtypeStruct((FLAT, D), jnp.float32),
        in_specs=[pl.BlockSpec(memory_space=pl.ANY)] * 5,
        out_specs=pl.BlockSpec(memory_space=pltpu.VMEM),
        scratch_shapes=[
            pltpu.VMEM((B, SQ, D), jnp.float32),
            pltpu.VMEM((D, HDL), jnp.float32),
            pltpu.VMEM((D, HDL), jnp.float32),
            pltpu.VMEM((D, HDL), jnp.float32),
            pltpu.VMEM((HDL, D), jnp.float32),
            pltpu.VMEM((FLAT, HDL), jnp.bfloat16),
            pltpu.VMEM((FLAT, D), jnp.bfloat16),
            pltpu.VMEM((3, FLAT, D), jnp.bfloat16),
            pltpu.SemaphoreType.DMA((5,)),
            pltpu.SemaphoreType.DMA((36,)),
            pltpu.SemaphoreType.DMA((36,)),
        ],
        compiler_params=pltpu.CompilerParams(collective_id=0),
    )(x, Wq, Wk, Wv, Wo)
    return flat.reshape(B, SQ, D)
